# baseline (speedup 1.0000x reference)
import functools

import jax
import jax.numpy as jnp
from jax import lax
from jax.experimental import pallas as pl
from jax.experimental.pallas import tpu as pltpu
from jax.experimental.pallas import tpu_sc as plsc

B = 16384
F = 26
FIELD_SIZE = 1000000
TABLE_ROWS = F * FIELD_SIZE
NC, NS, L = 2, 16, 16
NW = NC * NS
BPW = B // NW
CHUNK = 128
VPC = CHUNK // L
CPF = BPW // CHUNK

NSPLIT = 5


def _group_body(fh, xT, table, out, xv, idx_v, rows_v, out_v, sem):
    nchunk = fh * BPW // CHUNK
    wid = lax.axis_index("s") * NC + lax.axis_index("c")
    base = wid * BPW

    pltpu.sync_copy(xT.at[:, pl.ds(base, BPW)], xv)

    def fire(g, _):
        f = g // CPF
        part = g - f * CPF
        off = jnp.full((L,), f * FIELD_SIZE, jnp.int32)
        for j in range(VPC):
            idx_v[g, pl.ds(j * L, L)] = (
                xv[f, pl.ds(part * CHUNK + j * L, L)] + off
            )
        pltpu.make_async_copy(
            table.at[idx_v.at[g]], rows_v.at[pl.ds(g * CHUNK, CHUNK)], sem
        ).start()
        return 0

    lax.fori_loop(0, nchunk, fire, 0)

    def drain(g, _):
        pltpu.make_async_copy(
            table.at[idx_v.at[0]], rows_v.at[pl.ds(0, CHUNK)], sem
        ).wait()
        return 0

    lax.fori_loop(0, nchunk, drain, 0)

    def reduce_col(c, _):
        def inner(f, acc):
            return acc + rows_v[pl.ds(f * BPW + c * L, L)]

        out_v[pl.ds(c * L, L)] = lax.fori_loop(
            0, fh, inner, jnp.zeros((L,), jnp.float32)
        )
        return 0

    lax.fori_loop(0, BPW // L, reduce_col, 0)

    pltpu.sync_copy(out_v, out.at[pl.ds(base, BPW)])


@functools.lru_cache(maxsize=None)
def _group_kernel(fh):
    return pl.kernel(
        functools.partial(_group_body, fh),
        out_type=jax.ShapeDtypeStruct((B,), jnp.float32),
        mesh=plsc.VectorSubcoreMesh(core_axis_name="c", subcore_axis_name="s"),
        scratch_types=[
            pltpu.VMEM((fh, BPW), jnp.int32),
            pltpu.VMEM((fh * BPW // CHUNK, CHUNK), jnp.int32),
            pltpu.VMEM((fh * BPW,), jnp.float32),
            pltpu.VMEM((BPW,), jnp.float32),
            pltpu.SemaphoreType.DMA,
        ],
    )


def _combine_kernel(bias_ref, out_ref, *p_refs):
    s = p_refs[0][...] + bias_ref[0]
    for p in p_refs[1:]:
        s = s + p[...]
    out_ref[...] = 1.0 / (1.0 + jnp.exp(-s))


def _combine(partials, bias):
    def body(*refs):
        bias_ref, *p_refs, out_ref = refs
        _combine_kernel(bias_ref, out_ref, *p_refs)

    return pl.pallas_call(
        body,
        out_shape=jax.ShapeDtypeStruct((B,), jnp.float32),
    )(bias, *partials)


def kernel(x, table, bias):
    xT = x.T
    q, r = divmod(F, NSPLIT)
    counts = [q + 1] * r + [q] * (NSPLIT - r)
    partials, f0 = [], 0
    for fh in counts:
        tg = jnp.squeeze(table[f0 * FIELD_SIZE:(f0 + fh) * FIELD_SIZE], 1)
        partials.append(_group_kernel(fh)(xT[f0:f0 + fh], tg))
        f0 += fh
    return _combine(partials, bias.astype(jnp.float32))


# speedup vs baseline: 1.9711x; 1.9711x over previous
import functools

import jax
import jax.numpy as jnp
from jax import lax
from jax.experimental import pallas as pl
from jax.experimental.pallas import tpu as pltpu
from jax.experimental.pallas import tpu_sc as plsc

B = 16384
F = 26
FIELD_SIZE = 1000000
TABLE_ROWS = F * FIELD_SIZE
NC, NS, L = 2, 16, 16
NW = NC * NS
BPW = B // NW
CHUNK = 128
VPC = CHUNK // L
CPF = BPW // CHUNK

NSPLIT = 4


def _group_body(fh, xT, table, out, xv, idx_v, rows_v, out_v, sem):
    nchunk = fh * BPW // CHUNK
    wid = lax.axis_index("s") * NC + lax.axis_index("c")
    base = wid * BPW

    pltpu.sync_copy(xT.at[:, pl.ds(base, BPW)], xv)

    def fire(g, _):
        f = g // CPF
        part = g - f * CPF
        off = jnp.full((L,), f * FIELD_SIZE, jnp.int32)
        for j in range(VPC):
            idx_v[g, pl.ds(j * L, L)] = (
                xv[f, pl.ds(part * CHUNK + j * L, L)] + off
            )
        pltpu.make_async_copy(
            table.at[idx_v.at[g]], rows_v.at[pl.ds(g * CHUNK, CHUNK)], sem
        ).start()
        return 0

    lax.fori_loop(0, nchunk, fire, 0)

    def drain(g, _):
        pltpu.make_async_copy(
            table.at[idx_v.at[0]], rows_v.at[pl.ds(0, CHUNK)], sem
        ).wait()
        return 0

    lax.fori_loop(0, nchunk, drain, 0)

    def reduce_col(c, _):
        def inner(f, acc):
            return acc + rows_v[pl.ds(f * BPW + c * L, L)]

        out_v[pl.ds(c * L, L)] = lax.fori_loop(
            0, fh, inner, jnp.zeros((L,), jnp.float32)
        )
        return 0

    lax.fori_loop(0, BPW // L, reduce_col, 0)

    pltpu.sync_copy(out_v, out.at[pl.ds(base, BPW)])


@functools.lru_cache(maxsize=None)
def _group_kernel(fh):
    return pl.kernel(
        functools.partial(_group_body, fh),
        out_type=jax.ShapeDtypeStruct((B,), jnp.float32),
        mesh=plsc.VectorSubcoreMesh(core_axis_name="c", subcore_axis_name="s"),
        scratch_types=[
            pltpu.VMEM((fh, BPW), jnp.int32),
            pltpu.VMEM((fh * BPW // CHUNK, CHUNK), jnp.int32),
            pltpu.VMEM((fh * BPW,), jnp.float32),
            pltpu.VMEM((BPW,), jnp.float32),
            pltpu.SemaphoreType.DMA,
        ],
    )


def _combine_kernel(bias_ref, out_ref, *p_refs):
    s = p_refs[0][...] + bias_ref[0]
    for p in p_refs[1:]:
        s = s + p[...]
    out_ref[...] = 1.0 / (1.0 + jnp.exp(-s))


def _combine(partials, bias):
    def body(*refs):
        bias_ref, *p_refs, out_ref = refs
        _combine_kernel(bias_ref, out_ref, *p_refs)

    return pl.pallas_call(
        body,
        out_shape=jax.ShapeDtypeStruct((B,), jnp.float32),
    )(bias, *partials)


def kernel(x, table, bias):
    xT = x.T
    q, r = divmod(F, NSPLIT)
    counts = [q + 1] * r + [q] * (NSPLIT - r)
    partials, f0 = [], 0
    for fh in counts:
        tg = jnp.squeeze(table[f0 * FIELD_SIZE:(f0 + fh) * FIELD_SIZE], 1)
        partials.append(_group_kernel(fh)(xT[f0:f0 + fh], tg))
        f0 += fh
    return _combine(partials, bias.astype(jnp.float32))


# speedup vs baseline: 1.9774x; 1.0032x over previous
import functools

import jax
import jax.numpy as jnp
from jax import lax
from jax.experimental import pallas as pl
from jax.experimental.pallas import tpu as pltpu
from jax.experimental.pallas import tpu_sc as plsc

B = 16384
F = 26
FIELD_SIZE = 1000000
TABLE_ROWS = F * FIELD_SIZE
NC, NS, L = 2, 16, 16
NW = NC * NS
BPW = B // NW
CHUNK = 128
VPC = CHUNK // L
CPF = BPW // CHUNK

NSPLIT = 4


def _group_body(fh, xT, table, out, xv, idx_v, rows_v, out_v, sem):
    nchunk = fh * BPW // CHUNK
    wid = lax.axis_index("s") * NC + lax.axis_index("c")
    base = wid * BPW

    pltpu.sync_copy(xT.at[:, pl.ds(base, BPW)], xv)

    def fire(g, _):
        f = g // CPF
        part = g - f * CPF
        off = jnp.full((L,), f * FIELD_SIZE, jnp.int32)
        for j in range(VPC):
            idx_v[g, pl.ds(j * L, L)] = (
                xv[f, pl.ds(part * CHUNK + j * L, L)] + off
            )
        pltpu.make_async_copy(
            table.at[idx_v.at[g]], rows_v.at[pl.ds(g * CHUNK, CHUNK)], sem
        ).start()
        return 0

    lax.fori_loop(0, nchunk, fire, 0)

    def drain(g, _):
        pltpu.make_async_copy(
            table.at[idx_v.at[0]], rows_v.at[pl.ds(0, CHUNK)], sem
        ).wait()
        return 0

    lax.fori_loop(0, nchunk, drain, 0)

    def reduce_col(c, _):
        def inner(f, acc):
            return acc + rows_v[pl.ds(f * BPW + c * L, L)]

        out_v[pl.ds(c * L, L)] = lax.fori_loop(
            0, fh, inner, jnp.zeros((L,), jnp.float32)
        )
        return 0

    lax.fori_loop(0, BPW // L, reduce_col, 0)

    pltpu.sync_copy(out_v, out.at[pl.ds(base, BPW)])


@functools.lru_cache(maxsize=None)
def _group_kernel(fh):
    return pl.kernel(
        functools.partial(_group_body, fh),
        out_type=jax.ShapeDtypeStruct((B,), jnp.float32),
        mesh=plsc.VectorSubcoreMesh(core_axis_name="c", subcore_axis_name="s"),
        scratch_types=[
            pltpu.VMEM((fh, BPW), jnp.int32),
            pltpu.VMEM((fh * BPW // CHUNK, CHUNK), jnp.int32),
            pltpu.VMEM((fh * BPW,), jnp.float32),
            pltpu.VMEM((BPW,), jnp.float32),
            pltpu.SemaphoreType.DMA,
        ],
    )


def _combine_kernel(bias_ref, out_ref, *p_refs):
    s = p_refs[0][...] + bias_ref[0]
    for p in p_refs[1:]:
        s = s + p[...]
    out_ref[...] = 1.0 / (1.0 + jnp.exp(-s))


def _combine(partials, bias):
    def body(*refs):
        bias_ref, *p_refs, out_ref = refs
        _combine_kernel(bias_ref, out_ref, *p_refs)

    return pl.pallas_call(
        body,
        out_shape=jax.ShapeDtypeStruct((B,), jnp.float32),
    )(bias, *partials)


def kernel(x, table, bias):
    xT = x.T
    q, r = divmod(F, NSPLIT)
    counts = [q + 1] * r + [q] * (NSPLIT - r)
    partials, f0 = [], 0
    for fh in counts:
        tg = jnp.squeeze(table[f0 * FIELD_SIZE:(f0 + fh) * FIELD_SIZE], 1)
        tg = lax.optimization_barrier(tg)
        partials.append(_group_kernel(fh)(xT[f0:f0 + fh], tg))
        f0 += fh
    return _combine(partials, bias.astype(jnp.float32))
